# baseline (speedup 1.0000x reference)
import functools
from math import pi as PI

import jax
import jax.numpy as jnp
from jax import lax
from jax.experimental import pallas as pl
from jax.experimental.pallas import tpu as pltpu
from jax.experimental.pallas import tpu_sc as plsc

N_NODES = 10000
N_EDGES = 320000
HID = 256
FILT = 128
NUM_RADIAL = 32
NUM_BLOCKS = 4
NAT1 = 102
CUTOFF = 5.0
OUT = 256

NC, NS = 2, 16
NW = NC * NS
EPW = N_EDGES // NW
K = 80
NCHUNK = EPW // K
RPW = 624
RTAIL = N_NODES - RPW * NS

NBUFG = 3
NBUFP = 2
UNROLL = 6
NMAIN = (NCHUNK // UNROLL) * UNROLL

_mesh = plsc.VectorSubcoreMesh(core_axis_name="c", subcore_axis_name="s")


@functools.partial(
    pl.kernel,
    out_type=jax.ShapeDtypeStruct((N_EDGES,), jnp.float32),
    mesh=_mesh,
    scratch_types=[
        pltpu.VMEM((NBUFG, K), jnp.int32),
        pltpu.VMEM((NBUFG, K), jnp.int32),
        pltpu.VMEM((NBUFG, K), jnp.float32),
        pltpu.VMEM((NBUFG, K), jnp.float32),
        pltpu.VMEM((NBUFG, K), jnp.float32),
        pltpu.VMEM((NBUFG, K), jnp.float32),
        pltpu.VMEM((NBUFG, K), jnp.float32),
        pltpu.VMEM((NBUFG, K), jnp.float32),
        pltpu.VMEM((NBUFG, K), jnp.float32),
        pltpu.SemaphoreType.DMA,
        pltpu.SemaphoreType.DMA,
        pltpu.SemaphoreType.DMA,
        pltpu.SemaphoreType.DMA,
        pltpu.SemaphoreType.DMA,
        pltpu.SemaphoreType.DMA,
    ],
)
def _d2_sc(px, py, pz, src, dst, d2_out,
           srcv, dstv, xs, ys, zs, xd, yd, zd, acc,
           g0, g1, g2, st0, st1, st2):
    gsem = (g0, g1, g2)
    stsem = (st0, st1, st2)
    wid = lax.axis_index("s") * NC + lax.axis_index("c")
    base0 = wid * EPW

    def issue(t, base):
        pltpu.sync_copy(src.at[pl.ds(base, K)], srcv.at[t])
        pltpu.sync_copy(dst.at[pl.ds(base, K)], dstv.at[t])
        pltpu.async_copy(px.at[srcv.at[t]], xs.at[t], gsem[t])
        pltpu.async_copy(py.at[srcv.at[t]], ys.at[t], gsem[t])
        pltpu.async_copy(pz.at[srcv.at[t]], zs.at[t], gsem[t])
        pltpu.async_copy(px.at[dstv.at[t]], xd.at[t], gsem[t])
        pltpu.async_copy(py.at[dstv.at[t]], yd.at[t], gsem[t])
        pltpu.async_copy(pz.at[dstv.at[t]], zd.at[t], gsem[t])

    def drain_gather(t):
        pltpu.make_async_copy(px.at[srcv.at[t]], xs.at[t], gsem[t]).wait()
        pltpu.make_async_copy(py.at[srcv.at[t]], ys.at[t], gsem[t]).wait()
        pltpu.make_async_copy(pz.at[srcv.at[t]], zs.at[t], gsem[t]).wait()
        pltpu.make_async_copy(px.at[dstv.at[t]], xd.at[t], gsem[t]).wait()
        pltpu.make_async_copy(py.at[dstv.at[t]], yd.at[t], gsem[t]).wait()
        pltpu.make_async_copy(pz.at[dstv.at[t]], zd.at[t], gsem[t]).wait()

    def drain_store(t):
        pltpu.make_async_copy(acc.at[t], d2_out.at[pl.ds(base0, K)],
                              stsem[t]).wait()

    def process(c, t, static_tail):
        base = base0 + c * K
        if static_tail:
            drain_store(t)
        else:
            @pl.when(c >= NBUFG)
            def _():
                drain_store(t)
        drain_gather(t)
        for j in range(K // 16):
            sl = pl.ds(j * 16, 16)
            dx = xs[t, sl] - xd[t, sl]
            dy = ys[t, sl] - yd[t, sl]
            dz = zs[t, sl] - zd[t, sl]
            acc[t, sl] = dx * dx + dy * dy + dz * dz
        pltpu.async_copy(acc.at[t], d2_out.at[pl.ds(base, K)], stsem[t])
        nxt = c + NBUFG
        if static_tail:
            if nxt < NCHUNK:
                issue(t, base0 + nxt * K)
        else:
            issue(t, base0 + nxt * K)

    for t in range(NBUFG):
        issue(t, base0 + t * K)

    def body(j, carry):
        for u in range(NBUFG):
            process(j * NBUFG + u, u, False)
        return carry

    lax.fori_loop(0, NMAIN // NBUFG, body, 0)
    for c in range(NMAIN, NCHUNK):
        process(c, c % NBUFG, True)
    for t in range(NBUFG):
        drain_store(t)


@functools.partial(
    pl.kernel,
    out_type=jax.ShapeDtypeStruct((NC, N_NODES, FILT), jnp.float32),
    mesh=_mesh,
    scratch_types=[
        pltpu.VMEM((2, K), jnp.int32),
        pltpu.VMEM((2, K), jnp.int32),
        pltpu.VMEM((2, K, FILT), jnp.float32),
        pltpu.VMEM((2, K, FILT), jnp.float32),
        pltpu.VMEM((2, K), jnp.int32),
        pltpu.VMEM_SHARED((N_NODES, FILT), jnp.float32),
        pltpu.SemaphoreType.DMA,
        pltpu.SemaphoreType.DMA,
        pltpu.SemaphoreType.DMA,
        pltpu.SemaphoreType.DMA,
    ],
)
def _agg_sc(hh, wm, src, dst, zeros_nf, out,
            srcv, dstv, rows, wmv, pdst, agg_sh, g0, g1, s0, s1):
    gsem = (g0, g1)
    ssem = (s0, s1)
    c_ax = lax.axis_index("c")
    s_ax = lax.axis_index("s")
    wid = s_ax * NC + c_ax
    pltpu.sync_copy(zeros_nf.at[pl.ds(s_ax * RPW, RPW)],
                    agg_sh.at[pl.ds(s_ax * RPW, RPW)])

    @pl.when(s_ax == NS - 1)
    def _zero_tail():
        pltpu.sync_copy(zeros_nf.at[pl.ds(RPW * NS, RTAIL)],
                        agg_sh.at[pl.ds(RPW * NS, RTAIL)])

    plsc.subcore_barrier()
    base0 = wid * EPW

    def issue_rows(t, base):
        pltpu.sync_copy(src.at[pl.ds(base, K)], srcv.at[t])
        pltpu.sync_copy(dst.at[pl.ds(base, K)], dstv.at[t])
        pltpu.async_copy(hh.at[srcv.at[t]], rows.at[t], gsem[t])

    def issue_wm(t, base):
        pltpu.async_copy(wm.at[pl.ds(base, K)], wmv.at[t], gsem[t])

    def drain_gather(t, base):
        pltpu.make_async_copy(hh.at[srcv.at[t]], rows.at[t], gsem[t]).wait()
        pltpu.make_async_copy(wm.at[pl.ds(base, K)], wmv.at[t],
                              gsem[t]).wait()

    def drain_scatter(t):
        pltpu.make_async_copy(wmv.at[t], agg_sh.at[pdst.at[t]],
                              ssem[t]).wait()

    def process(c, t, static_tail):
        base = base0 + c * K
        drain_gather(t, base)
        for j in range(K // 16):
            sl = pl.ds(j * 16, 16)
            pdst[t, sl] = dstv[t, sl]

        def _mul(e, carry2):
            for j in range(FILT // 16):
                sl = pl.ds(j * 16, 16)
                wmv[t, e, sl] = rows[t, e, sl] * wmv[t, e, sl]
            return carry2

        lax.fori_loop(0, K, _mul, 0)
        pltpu.async_copy(wmv.at[t], agg_sh.at[pdst.at[t]], ssem[t],
                         add=True)
        nxt = c + 2
        if static_tail:
            if nxt < NCHUNK:
                issue_rows(t, base0 + nxt * K)
        else:
            issue_rows(t, base0 + nxt * K)
        if static_tail:
            if c + 1 < NCHUNK:
                drain_scatter(1 - t)
                issue_wm(1 - t, base0 + (c + 1) * K)
        else:
            @pl.when(c >= 1)
            def _():
                drain_scatter(1 - t)
            issue_wm(1 - t, base0 + (c + 1) * K)

    issue_rows(0, base0)
    issue_wm(0, base0)
    issue_rows(1, base0 + K)

    def body(j, carry):
        process(2 * j, 0, False)
        process(2 * j + 1, 1, False)
        return carry

    lax.fori_loop(0, (NCHUNK - 3) // 2, body, 0)
    for c in range(NCHUNK - 3, NCHUNK):
        process(c, c % 2, True)
    drain_scatter(1)
    drain_scatter(0)

    plsc.subcore_barrier()
    pltpu.sync_copy(agg_sh.at[pl.ds(s_ax * RPW, RPW)],
                    out.at[c_ax, pl.ds(s_ax * RPW, RPW)])

    @pl.when(s_ax == NS - 1)
    def _write_tail():
        pltpu.sync_copy(agg_sh.at[pl.ds(RPW * NS, RTAIL)],
                        out.at[c_ax, pl.ds(RPW * NS, RTAIL)])


_TN = 1000
_TE = 4000


def _emb_body(x_ref, emb_ref, wemb_ref, bemb_ref, wn_ref, wl_ref,
              h_ref, hh_ref):
    xv = x_ref[...]
    oh = (xv == lax.broadcasted_iota(jnp.int32, (_TN, NAT1), 1)
          ).astype(jnp.float32)
    m = jnp.dot(emb_ref[...], wemb_ref[...],
                preferred_element_type=jnp.float32)
    h = jnp.dot(oh, m, preferred_element_type=jnp.float32) + bemb_ref[...]
    h_ref[...] = h
    a = jnp.dot(wn_ref[...], wl_ref[...], preferred_element_type=jnp.float32)
    hh_ref[...] = jnp.dot(h, a, preferred_element_type=jnp.float32)


def _emb_call(x2, emb_table, W_emb, b_emb2, wn0, wl0):
    return pl.pallas_call(
        _emb_body,
        grid=(N_NODES // _TN,),
        in_specs=[
            pl.BlockSpec((_TN, 1), lambda i: (i, 0)),
            pl.BlockSpec((NAT1, 5), lambda i: (0, 0)),
            pl.BlockSpec((5, HID), lambda i: (0, 0)),
            pl.BlockSpec((1, HID), lambda i: (0, 0)),
            pl.BlockSpec((HID, FILT), lambda i: (0, 0)),
            pl.BlockSpec((FILT, FILT), lambda i: (0, 0)),
        ],
        out_specs=[
            pl.BlockSpec((_TN, HID), lambda i: (i, 0)),
            pl.BlockSpec((_TN, FILT), lambda i: (i, 0)),
        ],
        out_shape=[
            jax.ShapeDtypeStruct((N_NODES, HID), jnp.float32),
            jax.ShapeDtypeStruct((N_NODES, FILT), jnp.float32),
        ],
    )(x2, emb_table, W_emb, b_emb2, wn0, wl0)


def _wm_body(d2_ref, wr_ref, out_ref):
    d2 = d2_ref[...]
    d = jnp.sqrt(d2 + 1e-9)
    delta = CUTOFF / (NUM_RADIAL - 1)
    offs = lax.broadcasted_iota(jnp.int32, (_TE, NUM_RADIAL), 1
                                ).astype(jnp.float32) * delta
    coeff = -0.5 / (delta * delta)
    rbf = jnp.exp(coeff * (d - offs) ** 2)
    cenv = 0.5 * (jnp.cos(d * (PI / CUTOFF)) + 1.0)
    rbfc = rbf * cenv
    w4 = wr_ref[...]
    outs = [jnp.dot(rbfc, w4[b], preferred_element_type=jnp.float32)
            for b in range(NUM_BLOCKS)]
    out_ref[...] = jnp.stack(outs)


def _wm_call(d2col, W_r2m):
    return pl.pallas_call(
        _wm_body,
        grid=(N_EDGES // _TE,),
        in_specs=[
            pl.BlockSpec((_TE, 1), lambda i: (i, 0)),
            pl.BlockSpec((NUM_BLOCKS, NUM_RADIAL, FILT), lambda i: (0, 0, 0)),
        ],
        out_specs=pl.BlockSpec((NUM_BLOCKS, _TE, FILT), lambda i: (0, i, 0)),
        out_shape=jax.ShapeDtypeStruct((NUM_BLOCKS, N_EDGES, FILT),
                                       jnp.float32),
    )(d2col, W_r2m)


def _upd_body(h_ref, p_ref, wl2_ref, wm2n_ref, bl2_ref, wn_ref, wl1_ref,
              h_out, hh_out):
    p = p_ref[...]
    agg = p[0] + p[1]
    bmat = jnp.dot(wl2_ref[...], wm2n_ref[...],
                   preferred_element_type=jnp.float32)
    cvec = jnp.dot(bl2_ref[...], wm2n_ref[...],
                   preferred_element_type=jnp.float32)
    hn = h_ref[...] + jnp.dot(agg, bmat,
                              preferred_element_type=jnp.float32) + cvec
    h_out[...] = hn
    a = jnp.dot(wn_ref[...], wl1_ref[...], preferred_element_type=jnp.float32)
    hh_out[...] = jnp.dot(hn, a, preferred_element_type=jnp.float32)


def _upd_call(h, parts, wl2, wm2n, bl2row, wn_next, wl1_next):
    return pl.pallas_call(
        _upd_body,
        grid=(N_NODES // _TN,),
        in_specs=[
            pl.BlockSpec((_TN, HID), lambda i: (i, 0)),
            pl.BlockSpec((NC, _TN, FILT), lambda i: (0, i, 0)),
            pl.BlockSpec((FILT, FILT), lambda i: (0, 0)),
            pl.BlockSpec((FILT, HID), lambda i: (0, 0)),
            pl.BlockSpec((1, FILT), lambda i: (0, 0)),
            pl.BlockSpec((HID, FILT), lambda i: (0, 0)),
            pl.BlockSpec((FILT, FILT), lambda i: (0, 0)),
        ],
        out_specs=[
            pl.BlockSpec((_TN, HID), lambda i: (i, 0)),
            pl.BlockSpec((_TN, FILT), lambda i: (i, 0)),
        ],
        out_shape=[
            jax.ShapeDtypeStruct((N_NODES, HID), jnp.float32),
            jax.ShapeDtypeStruct((N_NODES, FILT), jnp.float32),
        ],
    )(h, parts, wl2, wm2n, bl2row, wn_next, wl1_next)


def _final_body(h_ref, p_ref, wl2_ref, wm2n_ref, bl2_ref, wout_ref, bout_ref,
                out_ref):
    p = p_ref[...]
    agg = p[0] + p[1]
    bmat = jnp.dot(wl2_ref[...], wm2n_ref[...],
                   preferred_element_type=jnp.float32)
    cvec = jnp.dot(bl2_ref[...], wm2n_ref[...],
                   preferred_element_type=jnp.float32)
    hn = h_ref[...] + jnp.dot(agg, bmat,
                              preferred_element_type=jnp.float32) + cvec
    out_ref[...] = jnp.dot(hn, wout_ref[...],
                           preferred_element_type=jnp.float32) + bout_ref[...]


def _final_call(h, parts, wl2, wm2n, bl2row, W_out, b_out2):
    return pl.pallas_call(
        _final_body,
        grid=(N_NODES // _TN,),
        in_specs=[
            pl.BlockSpec((_TN, HID), lambda i: (i, 0)),
            pl.BlockSpec((NC, _TN, FILT), lambda i: (0, i, 0)),
            pl.BlockSpec((FILT, FILT), lambda i: (0, 0)),
            pl.BlockSpec((FILT, HID), lambda i: (0, 0)),
            pl.BlockSpec((1, FILT), lambda i: (0, 0)),
            pl.BlockSpec((HID, OUT), lambda i: (0, 0)),
            pl.BlockSpec((1, OUT), lambda i: (0, 0)),
        ],
        out_specs=pl.BlockSpec((_TN, OUT), lambda i: (i, 0)),
        out_shape=jax.ShapeDtypeStruct((N_NODES, OUT), jnp.float32),
    )(h, parts, wl2, wm2n, bl2row, W_out, b_out2)


def kernel(x, pos, edge_index, batch, emb_table, W_emb, b_emb,
           W_n2m, W_r2m, W_lin1, W_lin2, b_lin2, W_m2n, W_out, b_out):
    src = edge_index[0].astype(jnp.int32)
    dst = edge_index[1].astype(jnp.int32)
    px = jnp.asarray(pos[:, 0], jnp.float32)
    py = jnp.asarray(pos[:, 1], jnp.float32)
    pz = jnp.asarray(pos[:, 2], jnp.float32)

    d2 = _d2_sc(px, py, pz, src, dst)
    d2col = d2.reshape(N_EDGES, 1)

    h, hh = _emb_call(x.reshape(N_NODES, 1).astype(jnp.int32),
                      emb_table, W_emb, b_emb.reshape(1, HID),
                      W_n2m[0], W_lin1[0])
    zeros_nf = jnp.zeros((N_NODES, FILT), jnp.float32)
    wm_all = _wm_call(d2col, W_r2m)
    out = None
    for b in range(NUM_BLOCKS):
        parts = _agg_sc(hh, wm_all[b], src, dst, zeros_nf)
        if b < NUM_BLOCKS - 1:
            h, hh = _upd_call(h, parts, W_lin2[b], W_m2n[b],
                              b_lin2[b].reshape(1, FILT),
                              W_n2m[b + 1], W_lin1[b + 1])
        else:
            out = _final_call(h, parts, W_lin2[b], W_m2n[b],
                              b_lin2[b].reshape(1, FILT),
                              W_out, b_out.reshape(1, OUT))
    return out


# speedup vs baseline: 1.1795x; 1.1795x over previous
import functools
from math import pi as PI

import jax
import jax.numpy as jnp
from jax import lax
from jax.experimental import pallas as pl
from jax.experimental.pallas import tpu as pltpu
from jax.experimental.pallas import tpu_sc as plsc

N_NODES = 10000
N_EDGES = 320000
HID = 256
FILT = 128
NUM_RADIAL = 32
NUM_BLOCKS = 4
NAT1 = 102
CUTOFF = 5.0
OUT = 256

NC, NS = 2, 16
NW = NC * NS
EPW = N_EDGES // NW
K = 80
NCHUNK = EPW // K
RPW = 624
RTAIL = N_NODES - RPW * NS

NBUFG = 3
NBUFP = 2
UNROLL = 6
NMAIN = (NCHUNK // UNROLL) * UNROLL

_mesh = plsc.VectorSubcoreMesh(core_axis_name="c", subcore_axis_name="s")


@functools.partial(
    pl.kernel,
    out_type=jax.ShapeDtypeStruct((N_EDGES,), jnp.float32),
    mesh=_mesh,
    scratch_types=[
        pltpu.VMEM((NBUFG, K), jnp.int32),
        pltpu.VMEM((NBUFG, K), jnp.int32),
        pltpu.VMEM((NBUFG, K), jnp.float32),
        pltpu.VMEM((NBUFG, K), jnp.float32),
        pltpu.VMEM((NBUFG, K), jnp.float32),
        pltpu.VMEM((NBUFG, K), jnp.float32),
        pltpu.VMEM((NBUFG, K), jnp.float32),
        pltpu.VMEM((NBUFG, K), jnp.float32),
        pltpu.VMEM((NBUFG, K), jnp.float32),
        pltpu.SemaphoreType.DMA,
        pltpu.SemaphoreType.DMA,
        pltpu.SemaphoreType.DMA,
        pltpu.SemaphoreType.DMA,
        pltpu.SemaphoreType.DMA,
        pltpu.SemaphoreType.DMA,
    ],
)
def _d2_sc(px, py, pz, src, dst, d2_out,
           srcv, dstv, xs, ys, zs, xd, yd, zd, acc,
           g0, g1, g2, st0, st1, st2):
    gsem = (g0, g1, g2)
    stsem = (st0, st1, st2)
    wid = lax.axis_index("s") * NC + lax.axis_index("c")
    base0 = wid * EPW

    def issue(t, base):
        pltpu.sync_copy(src.at[pl.ds(base, K)], srcv.at[t])
        pltpu.sync_copy(dst.at[pl.ds(base, K)], dstv.at[t])
        pltpu.async_copy(px.at[srcv.at[t]], xs.at[t], gsem[t])
        pltpu.async_copy(py.at[srcv.at[t]], ys.at[t], gsem[t])
        pltpu.async_copy(pz.at[srcv.at[t]], zs.at[t], gsem[t])
        pltpu.async_copy(px.at[dstv.at[t]], xd.at[t], gsem[t])
        pltpu.async_copy(py.at[dstv.at[t]], yd.at[t], gsem[t])
        pltpu.async_copy(pz.at[dstv.at[t]], zd.at[t], gsem[t])

    def drain_gather(t):
        pltpu.make_async_copy(px.at[srcv.at[t]], xs.at[t], gsem[t]).wait()
        pltpu.make_async_copy(py.at[srcv.at[t]], ys.at[t], gsem[t]).wait()
        pltpu.make_async_copy(pz.at[srcv.at[t]], zs.at[t], gsem[t]).wait()
        pltpu.make_async_copy(px.at[dstv.at[t]], xd.at[t], gsem[t]).wait()
        pltpu.make_async_copy(py.at[dstv.at[t]], yd.at[t], gsem[t]).wait()
        pltpu.make_async_copy(pz.at[dstv.at[t]], zd.at[t], gsem[t]).wait()

    def drain_store(t):
        pltpu.make_async_copy(acc.at[t], d2_out.at[pl.ds(base0, K)],
                              stsem[t]).wait()

    def process(c, t, static_tail):
        base = base0 + c * K
        if static_tail:
            drain_store(t)
        else:
            @pl.when(c >= NBUFG)
            def _():
                drain_store(t)
        drain_gather(t)
        for j in range(K // 16):
            sl = pl.ds(j * 16, 16)
            dx = xs[t, sl] - xd[t, sl]
            dy = ys[t, sl] - yd[t, sl]
            dz = zs[t, sl] - zd[t, sl]
            acc[t, sl] = dx * dx + dy * dy + dz * dz
        pltpu.async_copy(acc.at[t], d2_out.at[pl.ds(base, K)], stsem[t])
        nxt = c + NBUFG
        if static_tail:
            if nxt < NCHUNK:
                issue(t, base0 + nxt * K)
        else:
            issue(t, base0 + nxt * K)

    for t in range(NBUFG):
        issue(t, base0 + t * K)

    def body(j, carry):
        for u in range(NBUFG):
            process(j * NBUFG + u, u, False)
        return carry

    lax.fori_loop(0, NMAIN // NBUFG, body, 0)
    for c in range(NMAIN, NCHUNK):
        process(c, c % NBUFG, True)
    for t in range(NBUFG):
        drain_store(t)


@functools.partial(
    pl.kernel,
    out_type=jax.ShapeDtypeStruct((NC, N_NODES, FILT), jnp.float32),
    mesh=_mesh,
    scratch_types=[
        pltpu.VMEM((2, K), jnp.int32),
        pltpu.VMEM((2, K), jnp.int32),
        pltpu.VMEM((2, K, FILT), jnp.float32),
        pltpu.VMEM((2, K, FILT), jnp.float32),
        pltpu.VMEM_SHARED((N_NODES, FILT), jnp.float32),
        pltpu.SemaphoreType.DMA,
        pltpu.SemaphoreType.DMA,
    ],
)
def _agg_sc(hh, wm, src, dst, zeros_nf, out,
            srcv, dstv, rows, wmv, agg_sh, g0, g1):
    gsem = (g0, g1)
    c_ax = lax.axis_index("c")
    s_ax = lax.axis_index("s")
    wid = s_ax * NC + c_ax
    pltpu.sync_copy(zeros_nf.at[pl.ds(s_ax * RPW, RPW)],
                    agg_sh.at[pl.ds(s_ax * RPW, RPW)])

    @pl.when(s_ax == NS - 1)
    def _zero_tail():
        pltpu.sync_copy(zeros_nf.at[pl.ds(RPW * NS, RTAIL)],
                        agg_sh.at[pl.ds(RPW * NS, RTAIL)])

    plsc.subcore_barrier()
    base0 = wid * EPW

    def issue(t, base):
        pltpu.sync_copy(src.at[pl.ds(base, K)], srcv.at[t])
        pltpu.sync_copy(dst.at[pl.ds(base, K)], dstv.at[t])
        pltpu.async_copy(hh.at[srcv.at[t]], rows.at[t], gsem[t])
        pltpu.async_copy(wm.at[pl.ds(base, K)], wmv.at[t], gsem[t])

    def drain_gather(t, base):
        pltpu.make_async_copy(hh.at[srcv.at[t]], rows.at[t], gsem[t]).wait()
        pltpu.make_async_copy(wm.at[pl.ds(base, K)], wmv.at[t],
                              gsem[t]).wait()

    def process(c, t, static_tail):
        base = base0 + c * K
        drain_gather(t, base)

        def _mul(e, carry2):
            for j in range(FILT // 16):
                sl = pl.ds(j * 16, 16)
                rows[t, e, sl] = rows[t, e, sl] * wmv[t, e, sl]
            return carry2

        lax.fori_loop(0, K, _mul, 0)
        pltpu.sync_copy(rows.at[t], agg_sh.at[dstv.at[t]], add=True)
        nxt = c + 2
        if static_tail:
            if nxt < NCHUNK:
                issue(t, base0 + nxt * K)
        else:
            issue(t, base0 + nxt * K)

    issue(0, base0)
    issue(1, base0 + K)

    def body(j, carry):
        process(2 * j, 0, False)
        process(2 * j + 1, 1, False)
        return carry

    lax.fori_loop(0, (NCHUNK - 3) // 2, body, 0)
    for c in range(NCHUNK - 3, NCHUNK):
        process(c, c % 2, True)

    plsc.subcore_barrier()
    pltpu.sync_copy(agg_sh.at[pl.ds(s_ax * RPW, RPW)],
                    out.at[c_ax, pl.ds(s_ax * RPW, RPW)])

    @pl.when(s_ax == NS - 1)
    def _write_tail():
        pltpu.sync_copy(agg_sh.at[pl.ds(RPW * NS, RTAIL)],
                        out.at[c_ax, pl.ds(RPW * NS, RTAIL)])


_TN = 1000
_TE = 4000


def _emb_body(x_ref, emb_ref, wemb_ref, bemb_ref, wn_ref, wl_ref,
              h_ref, hh_ref):
    xv = x_ref[...]
    oh = (xv == lax.broadcasted_iota(jnp.int32, (_TN, NAT1), 1)
          ).astype(jnp.float32)
    m = jnp.dot(emb_ref[...], wemb_ref[...],
                preferred_element_type=jnp.float32)
    h = jnp.dot(oh, m, preferred_element_type=jnp.float32) + bemb_ref[...]
    h_ref[...] = h
    a = jnp.dot(wn_ref[...], wl_ref[...], preferred_element_type=jnp.float32)
    hh_ref[...] = jnp.dot(h, a, preferred_element_type=jnp.float32)


def _emb_call(x2, emb_table, W_emb, b_emb2, wn0, wl0):
    return pl.pallas_call(
        _emb_body,
        grid=(N_NODES // _TN,),
        in_specs=[
            pl.BlockSpec((_TN, 1), lambda i: (i, 0)),
            pl.BlockSpec((NAT1, 5), lambda i: (0, 0)),
            pl.BlockSpec((5, HID), lambda i: (0, 0)),
            pl.BlockSpec((1, HID), lambda i: (0, 0)),
            pl.BlockSpec((HID, FILT), lambda i: (0, 0)),
            pl.BlockSpec((FILT, FILT), lambda i: (0, 0)),
        ],
        out_specs=[
            pl.BlockSpec((_TN, HID), lambda i: (i, 0)),
            pl.BlockSpec((_TN, FILT), lambda i: (i, 0)),
        ],
        out_shape=[
            jax.ShapeDtypeStruct((N_NODES, HID), jnp.float32),
            jax.ShapeDtypeStruct((N_NODES, FILT), jnp.float32),
        ],
    )(x2, emb_table, W_emb, b_emb2, wn0, wl0)


def _wm_body(d2_ref, wr_ref, out_ref):
    d2 = d2_ref[...]
    d = jnp.sqrt(d2 + 1e-9)
    delta = CUTOFF / (NUM_RADIAL - 1)
    offs = lax.broadcasted_iota(jnp.int32, (_TE, NUM_RADIAL), 1
                                ).astype(jnp.float32) * delta
    coeff = -0.5 / (delta * delta)
    rbf = jnp.exp(coeff * (d - offs) ** 2)
    cenv = 0.5 * (jnp.cos(d * (PI / CUTOFF)) + 1.0)
    rbfc = rbf * cenv
    w4 = wr_ref[...]
    outs = [jnp.dot(rbfc, w4[b], preferred_element_type=jnp.float32)
            for b in range(NUM_BLOCKS)]
    out_ref[...] = jnp.stack(outs)


def _wm_call(d2col, W_r2m):
    return pl.pallas_call(
        _wm_body,
        grid=(N_EDGES // _TE,),
        in_specs=[
            pl.BlockSpec((_TE, 1), lambda i: (i, 0)),
            pl.BlockSpec((NUM_BLOCKS, NUM_RADIAL, FILT), lambda i: (0, 0, 0)),
        ],
        out_specs=pl.BlockSpec((NUM_BLOCKS, _TE, FILT), lambda i: (0, i, 0)),
        out_shape=jax.ShapeDtypeStruct((NUM_BLOCKS, N_EDGES, FILT),
                                       jnp.float32),
    )(d2col, W_r2m)


def _upd_body(h_ref, p_ref, wl2_ref, wm2n_ref, bl2_ref, wn_ref, wl1_ref,
              h_out, hh_out):
    p = p_ref[...]
    agg = p[0] + p[1]
    bmat = jnp.dot(wl2_ref[...], wm2n_ref[...],
                   preferred_element_type=jnp.float32)
    cvec = jnp.dot(bl2_ref[...], wm2n_ref[...],
                   preferred_element_type=jnp.float32)
    hn = h_ref[...] + jnp.dot(agg, bmat,
                              preferred_element_type=jnp.float32) + cvec
    h_out[...] = hn
    a = jnp.dot(wn_ref[...], wl1_ref[...], preferred_element_type=jnp.float32)
    hh_out[...] = jnp.dot(hn, a, preferred_element_type=jnp.float32)


def _upd_call(h, parts, wl2, wm2n, bl2row, wn_next, wl1_next):
    return pl.pallas_call(
        _upd_body,
        grid=(N_NODES // _TN,),
        in_specs=[
            pl.BlockSpec((_TN, HID), lambda i: (i, 0)),
            pl.BlockSpec((NC, _TN, FILT), lambda i: (0, i, 0)),
            pl.BlockSpec((FILT, FILT), lambda i: (0, 0)),
            pl.BlockSpec((FILT, HID), lambda i: (0, 0)),
            pl.BlockSpec((1, FILT), lambda i: (0, 0)),
            pl.BlockSpec((HID, FILT), lambda i: (0, 0)),
            pl.BlockSpec((FILT, FILT), lambda i: (0, 0)),
        ],
        out_specs=[
            pl.BlockSpec((_TN, HID), lambda i: (i, 0)),
            pl.BlockSpec((_TN, FILT), lambda i: (i, 0)),
        ],
        out_shape=[
            jax.ShapeDtypeStruct((N_NODES, HID), jnp.float32),
            jax.ShapeDtypeStruct((N_NODES, FILT), jnp.float32),
        ],
    )(h, parts, wl2, wm2n, bl2row, wn_next, wl1_next)


def _final_body(h_ref, p_ref, wl2_ref, wm2n_ref, bl2_ref, wout_ref, bout_ref,
                out_ref):
    p = p_ref[...]
    agg = p[0] + p[1]
    bmat = jnp.dot(wl2_ref[...], wm2n_ref[...],
                   preferred_element_type=jnp.float32)
    cvec = jnp.dot(bl2_ref[...], wm2n_ref[...],
                   preferred_element_type=jnp.float32)
    hn = h_ref[...] + jnp.dot(agg, bmat,
                              preferred_element_type=jnp.float32) + cvec
    out_ref[...] = jnp.dot(hn, wout_ref[...],
                           preferred_element_type=jnp.float32) + bout_ref[...]


def _final_call(h, parts, wl2, wm2n, bl2row, W_out, b_out2):
    return pl.pallas_call(
        _final_body,
        grid=(N_NODES // _TN,),
        in_specs=[
            pl.BlockSpec((_TN, HID), lambda i: (i, 0)),
            pl.BlockSpec((NC, _TN, FILT), lambda i: (0, i, 0)),
            pl.BlockSpec((FILT, FILT), lambda i: (0, 0)),
            pl.BlockSpec((FILT, HID), lambda i: (0, 0)),
            pl.BlockSpec((1, FILT), lambda i: (0, 0)),
            pl.BlockSpec((HID, OUT), lambda i: (0, 0)),
            pl.BlockSpec((1, OUT), lambda i: (0, 0)),
        ],
        out_specs=pl.BlockSpec((_TN, OUT), lambda i: (i, 0)),
        out_shape=jax.ShapeDtypeStruct((N_NODES, OUT), jnp.float32),
    )(h, parts, wl2, wm2n, bl2row, W_out, b_out2)


def kernel(x, pos, edge_index, batch, emb_table, W_emb, b_emb,
           W_n2m, W_r2m, W_lin1, W_lin2, b_lin2, W_m2n, W_out, b_out):
    src = edge_index[0].astype(jnp.int32)
    dst = edge_index[1].astype(jnp.int32)
    px = jnp.asarray(pos[:, 0], jnp.float32)
    py = jnp.asarray(pos[:, 1], jnp.float32)
    pz = jnp.asarray(pos[:, 2], jnp.float32)

    d2 = _d2_sc(px, py, pz, src, dst)
    d2col = d2.reshape(N_EDGES, 1)

    h, hh = _emb_call(x.reshape(N_NODES, 1).astype(jnp.int32),
                      emb_table, W_emb, b_emb.reshape(1, HID),
                      W_n2m[0], W_lin1[0])
    zeros_nf = jnp.zeros((N_NODES, FILT), jnp.float32)
    wm_all = _wm_call(d2col, W_r2m)
    out = None
    for b in range(NUM_BLOCKS):
        parts = _agg_sc(hh, wm_all[b], src, dst, zeros_nf)
        if b < NUM_BLOCKS - 1:
            h, hh = _upd_call(h, parts, W_lin2[b], W_m2n[b],
                              b_lin2[b].reshape(1, FILT),
                              W_n2m[b + 1], W_lin1[b + 1])
        else:
            out = _final_call(h, parts, W_lin2[b], W_m2n[b],
                              b_lin2[b].reshape(1, FILT),
                              W_out, b_out.reshape(1, OUT))
    return out
